# Initial kernel scaffold; baseline (speedup 1.0000x reference)
import jax
import jax.numpy as jnp
from jax.experimental import pallas as pl


def kernel(bseq, bseq_len, ei_o, ew_o, ei_p, ew_p, ei_n, ew_n, emb_basket, emb_item, gru_W_ih, gru_W_hh, gru_b_ih, gru_b_hh, lin_W, lin_b, ln_g, ln_b, W_bint, W_merge):
    raise NotImplementedError("write your pallas kernel here")


# speedup vs baseline: 2.6566x; 2.6566x over previous
import functools

import jax
import jax.numpy as jnp
from jax import lax
from jax.experimental import pallas as pl
from jax.experimental.pallas import tpu as pltpu, tpu_sc as plsc

NUM_BASKETS = 6000
NUM_ITEMS = 4000
N = NUM_BASKETS + NUM_ITEMS
E = 320000
D = 128
H = 128
NL = 3
NI = 4
B = 1024
L = 50

NC = 2
NS = 16
NW = NC * NS
NPAD = 10240
RPT = NPAD // NS
ZR = 160
EPT = E // NW
CK = 128
NFULL = EPT // CK
REM = EPT - NFULL * CK

_mesh = plsc.VectorSubcoreMesh(core_axis_name="c", subcore_axis_name="s")


def _zero_fill(buf, nrows):
    zero = jnp.zeros((16,), jnp.float32)

    def body(i, _):
        for j in range(D // 16):
            buf[i, pl.ds(j * 16, 16)] = zero
        return 0

    lax.fori_loop(0, nrows, body, 0)


def _scale_rows(rows, wv, nrows):
    def body(g, _):
        w16 = wv[pl.ds(g * 16, 16)]
        for l in range(16):
            w = w16[l]
            r = g * 16 + l
            for j in range(D // 16):
                sl = pl.ds(j * 16, 16)
                rows[r, sl] = rows[r, sl] * w
        return 0

    lax.fori_loop(0, nrows // 16, body, 0)


def _layer_body(src_ref, dst_ref, ew_ref, cur_ref, out_ref,
                sidx, didx, wv, rows, sidx2, didx2, wv2, zbuf, acc, sem):
    c = lax.axis_index("c")
    s = lax.axis_index("s")
    ebase = (c * NS + s) * EPT

    _zero_fill(zbuf, ZR)
    for t in range(RPT // ZR):
        pltpu.sync_copy(zbuf, acc.at[pl.ds(s * RPT + t * ZR, ZR)])
    plsc.subcore_barrier()

    def chunk(i, _):
        off = pl.multiple_of(ebase + i * CK, 8)
        pltpu.sync_copy(src_ref.at[pl.ds(off, CK)], sidx)
        pltpu.sync_copy(dst_ref.at[pl.ds(off, CK)], didx)
        pltpu.sync_copy(ew_ref.at[pl.ds(off, CK)], wv)
        pltpu.async_copy(cur_ref.at[sidx], rows, sem).wait()
        _scale_rows(rows, wv, CK)
        pltpu.sync_copy(rows, acc.at[didx], add=True)
        return 0

    lax.fori_loop(0, NFULL, chunk, 0)

    off = pl.multiple_of(ebase + NFULL * CK, 8)
    pltpu.sync_copy(src_ref.at[pl.ds(off, REM)], sidx2)
    pltpu.sync_copy(dst_ref.at[pl.ds(off, REM)], didx2)
    pltpu.sync_copy(ew_ref.at[pl.ds(off, REM)], wv2)
    rows_r = rows.at[pl.ds(0, REM)]
    pltpu.async_copy(cur_ref.at[sidx2], rows_r, sem).wait()
    _scale_rows(rows, wv2, REM)
    pltpu.sync_copy(rows_r, acc.at[didx2], add=True)

    plsc.subcore_barrier()
    sl = pl.ds(s * RPT, RPT)
    pltpu.sync_copy(acc.at[sl], out_ref.at[c, sl])


_layer = pl.kernel(
    _layer_body,
    out_type=jax.ShapeDtypeStruct((NC, NPAD, D), jnp.float32),
    mesh=_mesh,
    scratch_types=[
        pltpu.VMEM((CK,), jnp.int32),
        pltpu.VMEM((CK,), jnp.int32),
        pltpu.VMEM((CK,), jnp.float32),
        pltpu.VMEM((CK, D), jnp.float32),
        pltpu.VMEM((REM,), jnp.int32),
        pltpu.VMEM((REM,), jnp.int32),
        pltpu.VMEM((REM,), jnp.float32),
        pltpu.VMEM((ZR, D), jnp.float32),
        pltpu.VMEM_SHARED((NPAD, D), jnp.float32),
        pltpu.SemaphoreType.DMA,
    ],
)


def _merge_body(p_ref, sin_ref, cur_ref, sout_ref):
    curv = p_ref[0] + p_ref[1]
    cur_ref[...] = curv
    sout_ref[...] = sin_ref[...] + curv


_MB = 512


def _merge(partial, sum_in):
    grid = (NPAD // _MB,)
    return pl.pallas_call(
        _merge_body,
        grid=grid,
        in_specs=[
            pl.BlockSpec((NC, _MB, D), lambda i: (0, i, 0)),
            pl.BlockSpec((_MB, D), lambda i: (i, 0)),
        ],
        out_specs=[
            pl.BlockSpec((_MB, D), lambda i: (i, 0)),
            pl.BlockSpec((_MB, D), lambda i: (i, 0)),
        ],
        out_shape=[
            jax.ShapeDtypeStruct((NPAD, D), jnp.float32),
            jax.ShapeDtypeStruct((NPAD, D), jnp.float32),
        ],
    )(partial, sum_in)


def _prop_sum(src, dst, ew, all_emb):
    cur = all_emb
    ssum = all_emb
    for _ in range(NL):
        partial = _layer(src, dst, ew, cur)
        cur, ssum = _merge(partial, ssum)
    return ssum


def _gather_body(idx_ref, tab_ref, out_ref, idxv, rows, sem, *, rpt, ck):
    c = lax.axis_index("c")
    s = lax.axis_index("s")
    base = (c * NS + s) * rpt
    nfull = rpt // ck
    rem = rpt - nfull * ck

    def chunk(i, _):
        off = pl.multiple_of(base + i * ck, 8)
        pltpu.sync_copy(idx_ref.at[pl.ds(off, ck)], idxv)
        pltpu.async_copy(tab_ref.at[idxv], rows, sem).wait()
        pltpu.sync_copy(rows, out_ref.at[pl.ds(off, ck)])
        return 0

    lax.fori_loop(0, nfull, chunk, 0)
    if rem:
        off = pl.multiple_of(base + nfull * ck, 8)
        idx_r = idxv.at[pl.ds(0, rem)]
        rows_r = rows.at[pl.ds(0, rem)]
        pltpu.sync_copy(idx_ref.at[pl.ds(off, rem)], idx_r)
        pltpu.async_copy(tab_ref.at[idx_r], rows_r, sem).wait()
        pltpu.sync_copy(rows_r, out_ref.at[pl.ds(off, rem)])


def _make_gather(nrows):
    rpt = nrows // NW
    ck = min(CK, rpt)
    return pl.kernel(
        functools.partial(_gather_body, rpt=rpt, ck=ck),
        out_type=jax.ShapeDtypeStruct((nrows, D), jnp.float32),
        mesh=_mesh,
        scratch_types=[
            pltpu.VMEM((ck,), jnp.int32),
            pltpu.VMEM((ck, D), jnp.float32),
            pltpu.SemaphoreType.DMA,
        ],
    )


_seq_gather = _make_gather(B * L)


def _posneg_body(lastb_ref, pos_ref, neg_ref, pout_ref, nout_ref,
                 lbv, rows, sem):
    c = lax.axis_index("c")
    s = lax.axis_index("s")
    rpt = B // NW
    rbase = (c * NS + s) * rpt
    pltpu.sync_copy(lastb_ref.at[pl.ds(rbase, rpt)], lbv)
    pltpu.async_copy(pos_ref.at[lbv], rows, sem).wait()
    pltpu.sync_copy(rows, pout_ref.at[pl.ds(rbase, rpt)])
    pltpu.async_copy(neg_ref.at[lbv], rows, sem).wait()
    pltpu.sync_copy(rows, nout_ref.at[pl.ds(rbase, rpt)])


_posneg = pl.kernel(
    _posneg_body,
    out_type=[
        jax.ShapeDtypeStruct((B, D), jnp.float32),
        jax.ShapeDtypeStruct((B, D), jnp.float32),
    ],
    mesh=_mesh,
    scratch_types=[
        pltpu.VMEM((B // NW,), jnp.int32),
        pltpu.VMEM((B // NW, D), jnp.float32),
        pltpu.SemaphoreType.DMA,
    ],
)


def _lastb_body(bseq_ref, len_ref, out_ref):
    lens = len_ref[...]
    idx = jnp.minimum(jnp.maximum(lens, 1), L) - 1
    pos = lax.broadcasted_iota(jnp.int32, (B, L), 1)
    sel = jnp.where(pos == idx, bseq_ref[...], 0)
    out_ref[...] = jnp.sum(sel, axis=1, keepdims=True)


def _lastb(bseq, lens_b1):
    return pl.pallas_call(
        _lastb_body,
        out_shape=jax.ShapeDtypeStruct((B, 1), jnp.int32),
    )(bseq, lens_b1)


_BB = 256


def _gru_body(seq_ref, len_ref, wih_ref, whh_ref, bih_ref, bhh_ref,
              linw_ref, linb_ref, lng_ref, lnb_ref, wm_ref, out_ref):
    lens = len_ref[...]
    idx = jnp.minimum(jnp.maximum(lens, 1), L) - 1
    wih = wih_ref[...]
    whh = whh_ref[...]
    bih = bih_ref[...]
    bhh = bhh_ref[...]

    def step(t, carry):
        h, hl = carry
        x = seq_ref[pl.ds(t, 1)].reshape(_BB, D)
        gi = jnp.dot(x, wih, preferred_element_type=jnp.float32) + bih
        gh = jnp.dot(h, whh, preferred_element_type=jnp.float32) + bhh
        r = jax.nn.sigmoid(gi[:, :H] + gh[:, :H])
        z = jax.nn.sigmoid(gi[:, H:2 * H] + gh[:, H:2 * H])
        n = jnp.tanh(gi[:, 2 * H:] + r * gh[:, 2 * H:])
        h2 = (1.0 - z) * n + z * h
        hl2 = jnp.where(idx == t, h2, hl)
        return h2, hl2

    h0 = jnp.zeros((_BB, H), jnp.float32)
    _, hlast = lax.fori_loop(0, L, step, (h0, h0))

    x = jnp.dot(hlast, linw_ref[...], preferred_element_type=jnp.float32)
    x = x + linb_ref[...]
    mu = jnp.mean(x, axis=-1, keepdims=True)
    xc = x - mu
    var = jnp.mean(xc * xc, axis=-1, keepdims=True)
    x = xc * lax.rsqrt(var + 1e-12) * lng_ref[...] + lnb_ref[...]
    out_ref[...] = jnp.dot(x, wm_ref[...], preferred_element_type=jnp.float32)


def _gru_head(seq_lbd, lens_b1, wih, whh, bih, bhh, linw, linb, lng, lnb, wm):
    grid = (B // _BB,)
    full = lambda shape: pl.BlockSpec(shape, lambda i: tuple(0 for _ in shape))
    return pl.pallas_call(
        _gru_body,
        grid=grid,
        in_specs=[
            pl.BlockSpec((L, _BB, D), lambda i: (0, i, 0)),
            pl.BlockSpec((_BB, 1), lambda i: (i, 0)),
            full((D, 3 * H)),
            full((H, 3 * H)),
            full((1, 3 * H)),
            full((1, 3 * H)),
            full((H, D)),
            full((1, D)),
            full((1, D)),
            full((1, D)),
            full((D, D)),
        ],
        out_specs=pl.BlockSpec((_BB, D), lambda i: (i, 0)),
        out_shape=jax.ShapeDtypeStruct((B, D), jnp.float32),
    )(seq_lbd, lens_b1, wih, whh, bih, bhh, linw, linb, lng, lnb, wm)


def _scores_body(m_ref, p_ref, n_ref, it_ref, out_ref):
    m = m_ref[...] + 0.0025 * (p_ref[...] - n_ref[...])
    out_ref[...] = 0.25 * lax.dot_general(
        m, it_ref[...], (((1,), (1,)), ((), ())),
        preferred_element_type=jnp.float32)


def _scores(merged, posr, negr, items):
    return pl.pallas_call(
        _scores_body,
        out_shape=jax.ShapeDtypeStruct((B, NUM_ITEMS), jnp.float32),
    )(merged, posr, negr, items)


def kernel(bseq, bseq_len, ei_o, ew_o, ei_p, ew_p, ei_n, ew_n, emb_basket,
           emb_item, gru_W_ih, gru_W_hh, gru_b_ih, gru_b_hh, lin_W, lin_b,
           ln_g, ln_b, W_bint, W_merge):
    all_emb = jnp.concatenate([emb_basket, emb_item], axis=0)
    all_emb = jnp.pad(all_emb, ((0, NPAD - N), (0, 0)))

    sum_o = _prop_sum(ei_o[0], ei_o[1], ew_o, all_emb)
    sum_p = _prop_sum(ei_p[0], ei_p[1], ew_p, all_emb)
    sum_n = _prop_sum(ei_n[0], ei_n[1], ew_n, all_emb)

    bseq_t_flat = bseq.T.reshape(-1)
    seq = _seq_gather(bseq_t_flat, sum_o)
    seq_lbd = seq.reshape(L, B, D)

    lastb = _lastb(bseq, bseq_len.reshape(B, 1)).reshape(B)
    posr, negr = _posneg(lastb, sum_p, sum_n)

    wih = gru_W_ih.T * 0.25
    whh = gru_W_hh.T
    bih = gru_b_ih.reshape(1, 3 * H)
    bhh = gru_b_hh.reshape(1, 3 * H)
    linw = lin_W.T
    linb = lin_b.reshape(1, D)
    lng = ln_g.reshape(1, D)
    lnb = ln_b.reshape(1, D)
    wm = (W_bint.reshape(NI, D, D) * W_merge[0][:, None, None]).sum(0).T

    merged = _gru_head(seq_lbd, bseq_len.reshape(B, 1), wih, whh, bih, bhh,
                       linw, linb, lng, lnb, wm)

    items = lax.slice(sum_o, (NUM_BASKETS, 0), (N, D))
    return _scores(merged, posr, negr, items)
